# baseline (speedup 1.0000x reference)
import functools

import jax
import jax.numpy as jnp
from jax import lax
from jax.experimental import pallas as pl
from jax.experimental.pallas import tpu as pltpu
from jax.experimental.pallas import tpu_sc as plsc

B = 16384
L = 200
DIM = 64
LABELS = 10
VOCAB = 100000
LP = 16

NC = 2
NS = 16
NW = NC * NS
ROWS_PER_W = B // NW
CB = 16
N_CHUNKS = ROWS_PER_W // CB
FLAT = CB * L
L0, L1 = 104, 96
UNROLL = 25
NACC = 4


def _issue_gathers(t2_hbm, idx_v, rows_v, sem):
    copies = []
    for j in range(CB):
        copies.append(pltpu.async_copy(
            t2_hbm.at[idx_v.at[j, pl.ds(0, L0)]],
            rows_v.at[pl.ds(j * L, L0)], sem))
        copies.append(pltpu.async_copy(
            t2_hbm.at[idx_v.at[j, pl.ds(L0, L1)]],
            rows_v.at[pl.ds(j * L + L0, L1)], sem))
    return copies


def _wait_gathers(t2_hbm, idx_v, rows_v, sem):
    for j in range(CB):
        pltpu.make_async_copy(
            t2_hbm.at[idx_v.at[j, pl.ds(0, L0)]],
            rows_v.at[pl.ds(j * L, L0)], sem).wait()
        pltpu.make_async_copy(
            t2_hbm.at[idx_v.at[j, pl.ds(L0, L1)]],
            rows_v.at[pl.ds(j * L + L0, L1)], sem).wait()


def _accumulate_softmax(rows_v, out_v, b12):
    for j in range(CB):
        j0 = j * L

        def body(r, accs, j0=j0):
            r0 = j0 + r * UNROLL
            accs = list(accs)
            for u in range(UNROLL):
                accs[u % NACC] = accs[u % NACC] + rows_v[r0 + u, :]
            return tuple(accs)
        accs = lax.fori_loop(
            0, L // UNROLL, body,
            tuple(jnp.zeros((LP,), jnp.float32) for _ in range(NACC)))
        z = (accs[0] + accs[1]) + (accs[2] + accs[3]) + b12
        e = jnp.exp(z - jnp.max(z))
        out_v[j, :] = e / lax.broadcast(jnp.sum(e), (LP,))


def _sc_pool_kernel(t2_hbm, text_hbm, b12_hbm, out_hbm,
                    idx0, idx1, rows0, rows1, out0, out1, b12_v,
                    sem0, sem1, semw0, semw1, semt0, semt1):
    wid = lax.axis_index("s") * NC + lax.axis_index("c")
    base = wid * ROWS_PER_W
    idx = (idx0, idx1)
    rows = (rows0, rows1)
    outs = (out0, out1)
    semg = (sem0, sem1)
    semw = (semw0, semw1)
    semt = (semt0, semt1)

    pltpu.sync_copy(b12_hbm, b12_v)
    b12 = b12_v[...]

    pltpu.sync_copy(text_hbm.at[pl.ds(base, CB)], idx0)
    _issue_gathers(t2_hbm, idx0, rows0, sem0)
    pltpu.async_copy(text_hbm.at[pl.ds(base + CB, CB)], idx1, semt1)

    def stage(c, k, p):
        q = 1 - p

        def _launch_next():
            pltpu.make_async_copy(
                text_hbm.at[pl.ds(base, CB)], idx[q], semt[q]).wait()
            _issue_gathers(t2_hbm, idx[q], rows[q], semg[q])
        if p == 0:
            _launch_next()
        else:
            pl.when(k < N_CHUNKS // 2 - 1)(_launch_next)
        _wait_gathers(t2_hbm, idx[p], rows[p], semg[p])
        @pl.when(k < N_CHUNKS // 2 - 1)
        def _():
            pltpu.async_copy(text_hbm.at[pl.ds(base + (c + 2) * CB, CB)],
                             idx[p], semt[p])
        @pl.when(k > 0)
        def _():
            pltpu.make_async_copy(
                outs[p], out_hbm.at[pl.ds(base, CB)], semw[p]).wait()
        _accumulate_softmax(rows[p], outs[p], b12)
        pltpu.async_copy(outs[p], out_hbm.at[pl.ds(base + c * CB, CB)],
                         semw[p])

    def body(k, _):
        stage(2 * k, k, 0)
        stage(2 * k + 1, k, 1)
        return ()

    lax.fori_loop(0, N_CHUNKS // 2, body, ())
    pltpu.make_async_copy(out0, out_hbm.at[pl.ds(base, CB)], semw0).wait()
    pltpu.make_async_copy(out1, out_hbm.at[pl.ds(base, CB)], semw1).wait()


@jax.jit
def _sc_pool(t2, text, b12):
    mesh = plsc.VectorSubcoreMesh(core_axis_name="c", subcore_axis_name="s")
    return pl.kernel(
        _sc_pool_kernel,
        out_type=jax.ShapeDtypeStruct((B, LP), jnp.float32),
        mesh=mesh,
        compiler_params=pltpu.CompilerParams(use_tc_tiling_on_sc=False,
                                             needs_layout_passes=False),
        scratch_types=[
            pltpu.VMEM((CB, L), jnp.int32),
            pltpu.VMEM((CB, L), jnp.int32),
            pltpu.VMEM((FLAT, LP), jnp.float32),
            pltpu.VMEM((FLAT, LP), jnp.float32),
            pltpu.VMEM((CB, LP), jnp.float32),
            pltpu.VMEM((CB, LP), jnp.float32),
            pltpu.VMEM((LP,), jnp.float32),
            pltpu.SemaphoreType.DMA,
            pltpu.SemaphoreType.DMA,
            pltpu.SemaphoreType.DMA,
            pltpu.SemaphoreType.DMA,
            pltpu.SemaphoreType.DMA,
            pltpu.SemaphoreType.DMA,
        ],
    )(t2, text, b12)


def _precompute_kernel(table_ref, W1_ref, W2p_ref, b1_ref, b2p_ref,
                       t2_ref, b12_ref):
    w12 = jnp.dot(W1_ref[...], W2p_ref[...],
                  preferred_element_type=jnp.float32,
                  precision=lax.Precision.HIGHEST)
    t2_ref[...] = jnp.dot(table_ref[...], w12,
                          preferred_element_type=jnp.float32) * (1.0 / L)
    b12_ref[...] = jnp.dot(b1_ref[...], W2p_ref[...],
                           preferred_element_type=jnp.float32,
                           precision=lax.Precision.HIGHEST) + b2p_ref[...]


@jax.jit
def _precompute(table, W1, W2p, b1, b2p):
    NBLK = 10
    BLKV = VOCAB // NBLK
    return pl.pallas_call(
        _precompute_kernel,
        grid=(NBLK,),
        in_specs=[
            pl.BlockSpec((BLKV, DIM), lambda i: (i, 0)),
            pl.BlockSpec((DIM, DIM), lambda i: (0, 0)),
            pl.BlockSpec((DIM, LP), lambda i: (0, 0)),
            pl.BlockSpec((1, DIM), lambda i: (0, 0)),
            pl.BlockSpec((1, LP), lambda i: (0, 0)),
        ],
        out_specs=[
            pl.BlockSpec((BLKV, LP), lambda i: (i, 0)),
            pl.BlockSpec((1, LP), lambda i: (0, 0)),
        ],
        out_shape=[
            jax.ShapeDtypeStruct((VOCAB, LP), jnp.float32),
            jax.ShapeDtypeStruct((1, LP), jnp.float32),
        ],
    )(table, W1, W2p, b1, b2p)


def kernel(text, text_lengths, table, W1, b1, W2, b2):
    del text_lengths
    text = text.astype(jnp.int32)
    W2p = jnp.pad(W2, ((0, 0), (0, LP - LABELS)))
    b2p = jnp.full((1, LP), -1e30, jnp.float32).at[0, :LABELS].set(b2)
    t2, b12 = _precompute(table, W1, W2p, b1.reshape(1, DIM), b2p)
    probs = _sc_pool(t2, text, b12.reshape(LP))
    return probs[:, :LABELS]


# speedup vs baseline: 1.1743x; 1.1743x over previous
import functools

import jax
import jax.numpy as jnp
from jax import lax
from jax.experimental import pallas as pl
from jax.experimental.pallas import tpu as pltpu
from jax.experimental.pallas import tpu_sc as plsc

B = 16384
L = 200
DIM = 64
LABELS = 10
VOCAB = 100000
LP = 16

NC = 2
NS = 16
NW = NC * NS
ROWS_PER_W = B // NW
CB = 16
N_CHUNKS = ROWS_PER_W // CB
FLAT = CB * L
L0, L1 = 104, 96
UNROLL = 8
NACC = 2


def _issue_gathers(t2_hbm, idx_v, rows_v, sem):
    copies = []
    for j in range(CB):
        copies.append(pltpu.async_copy(
            t2_hbm.at[idx_v.at[j, pl.ds(0, L0)]],
            rows_v.at[pl.ds(j * L, L0)], sem))
        copies.append(pltpu.async_copy(
            t2_hbm.at[idx_v.at[j, pl.ds(L0, L1)]],
            rows_v.at[pl.ds(j * L + L0, L1)], sem))
    return copies


def _wait_gathers(t2_hbm, idx_v, rows_v, sem):
    for j in range(CB):
        pltpu.make_async_copy(
            t2_hbm.at[idx_v.at[j, pl.ds(0, L0)]],
            rows_v.at[pl.ds(j * L, L0)], sem).wait()
        pltpu.make_async_copy(
            t2_hbm.at[idx_v.at[j, pl.ds(L0, L1)]],
            rows_v.at[pl.ds(j * L + L0, L1)], sem).wait()


def _accumulate_softmax(rows_v, out_v, b12):
    for j in range(CB):
        j0 = j * L

        def body(r, accs, j0=j0):
            r0 = j0 + r * UNROLL
            accs = list(accs)
            for u in range(UNROLL):
                accs[u % NACC] = accs[u % NACC] + rows_v[r0 + u, :]
            return tuple(accs)
        accs = lax.fori_loop(
            0, L // UNROLL, body,
            tuple(jnp.zeros((LP,), jnp.float32) for _ in range(NACC)))
        z = sum(accs[1:], accs[0]) + b12
        e = jnp.exp(z - jnp.max(z))
        out_v[j, :] = e / lax.broadcast(jnp.sum(e), (LP,))


def _sc_pool_kernel(t2_hbm, text_hbm, b12_hbm, out_hbm,
                    idx0, idx1, rows0, rows1, out0, out1, b12_v,
                    sem0, sem1, semw0, semw1, semt0, semt1):
    wid = lax.axis_index("s") * NC + lax.axis_index("c")
    base = wid * ROWS_PER_W
    idx = (idx0, idx1)
    rows = (rows0, rows1)
    outs = (out0, out1)
    semg = (sem0, sem1)
    semw = (semw0, semw1)
    semt = (semt0, semt1)

    pltpu.sync_copy(b12_hbm, b12_v)
    b12 = b12_v[...]

    pltpu.sync_copy(text_hbm.at[pl.ds(base, CB)], idx0)
    _issue_gathers(t2_hbm, idx0, rows0, sem0)
    pltpu.async_copy(text_hbm.at[pl.ds(base + CB, CB)], idx1, semt1)

    def stage(c, k, p):
        q = 1 - p

        def _launch_next():
            pltpu.make_async_copy(
                text_hbm.at[pl.ds(base, CB)], idx[q], semt[q]).wait()
            _issue_gathers(t2_hbm, idx[q], rows[q], semg[q])
        if p == 0:
            _launch_next()
        else:
            pl.when(k < N_CHUNKS // 2 - 1)(_launch_next)
        _wait_gathers(t2_hbm, idx[p], rows[p], semg[p])
        @pl.when(k < N_CHUNKS // 2 - 1)
        def _():
            pltpu.async_copy(text_hbm.at[pl.ds(base + (c + 2) * CB, CB)],
                             idx[p], semt[p])
        @pl.when(k > 0)
        def _():
            pltpu.make_async_copy(
                outs[p], out_hbm.at[pl.ds(base, CB)], semw[p]).wait()
        _accumulate_softmax(rows[p], outs[p], b12)
        pltpu.async_copy(outs[p], out_hbm.at[pl.ds(base + c * CB, CB)],
                         semw[p])

    def body(k, _):
        stage(2 * k, k, 0)
        stage(2 * k + 1, k, 1)
        return ()

    lax.fori_loop(0, N_CHUNKS // 2, body, ())
    pltpu.make_async_copy(out0, out_hbm.at[pl.ds(base, CB)], semw0).wait()
    pltpu.make_async_copy(out1, out_hbm.at[pl.ds(base, CB)], semw1).wait()


@jax.jit
def _sc_pool(t2, text, b12):
    mesh = plsc.VectorSubcoreMesh(core_axis_name="c", subcore_axis_name="s")
    return pl.kernel(
        _sc_pool_kernel,
        out_type=jax.ShapeDtypeStruct((B, LP), jnp.float32),
        mesh=mesh,
        compiler_params=pltpu.CompilerParams(use_tc_tiling_on_sc=False,
                                             needs_layout_passes=False),
        scratch_types=[
            pltpu.VMEM((CB, L), jnp.int32),
            pltpu.VMEM((CB, L), jnp.int32),
            pltpu.VMEM((FLAT, LP), jnp.float32),
            pltpu.VMEM((FLAT, LP), jnp.float32),
            pltpu.VMEM((CB, LP), jnp.float32),
            pltpu.VMEM((CB, LP), jnp.float32),
            pltpu.VMEM((LP,), jnp.float32),
            pltpu.SemaphoreType.DMA,
            pltpu.SemaphoreType.DMA,
            pltpu.SemaphoreType.DMA,
            pltpu.SemaphoreType.DMA,
            pltpu.SemaphoreType.DMA,
            pltpu.SemaphoreType.DMA,
        ],
    )(t2, text, b12)


def _precompute_kernel(tableT_ref, W1_ref, W2p_ref, b1_ref, b2p_ref,
                       t2_ref, b12_ref):
    w12 = jnp.dot(W1_ref[...], W2p_ref[...],
                  preferred_element_type=jnp.float32,
                  precision=lax.Precision.HIGHEST)
    t2_ref[...] = lax.dot_general(
        tableT_ref[...], w12, (((0,), (0,)), ((), ())),
        preferred_element_type=jnp.float32) * (1.0 / L)
    b12_ref[...] = jnp.dot(b1_ref[...], W2p_ref[...],
                           preferred_element_type=jnp.float32,
                           precision=lax.Precision.HIGHEST) + b2p_ref[...]


@jax.jit
def _precompute(tableT, W1, W2p, b1, b2p):
    BLKV = 12800
    NBLK = (VOCAB + BLKV - 1) // BLKV
    return pl.pallas_call(
        _precompute_kernel,
        grid=(NBLK,),
        in_specs=[
            pl.BlockSpec((DIM, BLKV), lambda i: (0, i)),
            pl.BlockSpec((DIM, DIM), lambda i: (0, 0)),
            pl.BlockSpec((DIM, LP), lambda i: (0, 0)),
            pl.BlockSpec((1, DIM), lambda i: (0, 0)),
            pl.BlockSpec((1, LP), lambda i: (0, 0)),
        ],
        out_specs=[
            pl.BlockSpec((BLKV, LP), lambda i: (i, 0)),
            pl.BlockSpec((1, LP), lambda i: (0, 0)),
        ],
        out_shape=[
            jax.ShapeDtypeStruct((VOCAB, LP), jnp.float32),
            jax.ShapeDtypeStruct((1, LP), jnp.float32),
        ],
    )(tableT, W1, W2p, b1, b2p)


def kernel(text, text_lengths, table, W1, b1, W2, b2):
    del text_lengths
    text = text.astype(jnp.int32)
    W2p = jnp.pad(W2, ((0, 0), (0, LP - LABELS)))
    b2p = jnp.full((1, LP), -1e30, jnp.float32).at[0, :LABELS].set(b2)
    t2, b12 = _precompute(table.T, W1, W2p, b1.reshape(1, DIM), b2p)
    probs = _sc_pool(t2, text, b12.reshape(LP))
    return probs[:, :LABELS]
